# baseline (speedup 1.0000x reference)
import functools

import jax
import jax.numpy as jnp
from jax import lax
from jax.experimental import pallas as pl
from jax.experimental.pallas import tpu as pltpu
from jax.experimental.pallas import tpu_sc as plsc

L = 200
WORD_DIM = 64
POS_DIM = 64
HIDDEN = WORD_DIM + POS_DIM
ROWS_PER_WORKER = 8
NUM_CHUNKS = L // ROWS_PER_WORKER

_info = plsc.get_sparse_core_info()
_NC = _info.num_cores


def _make_gather():
    mesh = plsc.VectorSubcoreMesh(core_axis_name="c", subcore_axis_name="s")

    @functools.partial(
        pl.kernel,
        mesh=mesh,
        out_type=jax.ShapeDtypeStruct((L, WORD_DIM), jnp.float32),
        compiler_params=pltpu.CompilerParams(use_tc_tiling_on_sc=False),
        scratch_types=[
            pltpu.VMEM((ROWS_PER_WORKER,), jnp.int32),
            pltpu.VMEM((ROWS_PER_WORKER, WORD_DIM), jnp.float32),
            pltpu.SemaphoreType.DMA,
        ],
    )
    def gather(idx_hbm, table_hbm, out_hbm, idx_v, rows_v, sem):
        wid = lax.axis_index("s") * _NC + lax.axis_index("c")

        @pl.when(wid < NUM_CHUNKS)
        def _():
            base = wid * ROWS_PER_WORKER
            pltpu.sync_copy(idx_hbm.at[pl.ds(base, ROWS_PER_WORKER)], idx_v)
            pltpu.async_copy(table_hbm.at[idx_v], rows_v, sem).wait()
            pltpu.sync_copy(rows_v, out_hbm.at[pl.ds(base, ROWS_PER_WORKER)])

    return gather


_gather = _make_gather()


def _assemble_body(word_ref, pos_ref, out_ref, hid_ref):
    rows = jnp.concatenate([word_ref[...], pos_ref[...]], axis=1)
    out_ref[...] = rows
    hid_ref[...] = jnp.mean(rows, axis=0).reshape(1, 1, HIDDEN)


_assemble = pl.pallas_call(
    _assemble_body,
    out_shape=(
        jax.ShapeDtypeStruct((L, HIDDEN), jnp.float32),
        jax.ShapeDtypeStruct((1, 1, HIDDEN), jnp.float32),
    ),
)


@jax.jit
def kernel(input, W_word, W_pos):
    idx = input.astype(jnp.int32)
    word = _gather(idx, W_word)
    return _assemble(word, W_pos)


# speedup vs baseline: 1.7367x; 1.7367x over previous
import functools

import jax
import jax.numpy as jnp
from jax import lax
from jax.experimental import pallas as pl
from jax.experimental.pallas import tpu as pltpu
from jax.experimental.pallas import tpu_sc as plsc

L = 200
WORD_DIM = 64
POS_DIM = 64
HIDDEN = WORD_DIM + POS_DIM
ROWS_PER_WORKER = 8
NUM_CHUNKS = L // ROWS_PER_WORKER

_info = plsc.get_sparse_core_info()
_NC = _info.num_cores
_NLANES = _info.num_lanes


def _make_gather():
    mesh = plsc.VectorSubcoreMesh(core_axis_name="c", subcore_axis_name="s")

    @functools.partial(
        pl.kernel,
        mesh=mesh,
        out_type=jax.ShapeDtypeStruct((L, WORD_DIM), jnp.float32),
        scratch_types=[
            pltpu.VMEM((_NLANES,), jnp.int32),
            pltpu.VMEM((ROWS_PER_WORKER, WORD_DIM), jnp.float32),
            pltpu.SemaphoreType.DMA,
        ],
    )
    def gather(idx_hbm, table_hbm, out_hbm, idx_v, rows_v, sem):
        wid = lax.axis_index("s") * _NC + lax.axis_index("c")

        @pl.when(wid < NUM_CHUNKS)
        def _():
            base = wid * ROWS_PER_WORKER
            pltpu.sync_copy(
                idx_hbm.at[pl.ds(base, ROWS_PER_WORKER)],
                idx_v.at[pl.ds(0, ROWS_PER_WORKER)],
            )
            idx_vec = idx_v[...]
            copies = []
            for r in range(ROWS_PER_WORKER):
                row_idx = idx_vec[r]
                copies.append(
                    pltpu.make_async_copy(
                        table_hbm.at[row_idx], rows_v.at[r], sem
                    )
                )
            for c in copies:
                c.start()
            for c in copies:
                c.wait()
            pltpu.sync_copy(rows_v, out_hbm.at[pl.ds(base, ROWS_PER_WORKER)])

    return gather


_gather = _make_gather()


def _assemble_body(word_ref, pos_ref, out_ref, hid_ref):
    rows = jnp.concatenate([word_ref[...], pos_ref[...]], axis=1)
    out_ref[...] = rows
    hid_ref[...] = jnp.mean(rows, axis=0).reshape(1, 1, HIDDEN)


_assemble = pl.pallas_call(
    _assemble_body,
    out_shape=(
        jax.ShapeDtypeStruct((L, HIDDEN), jnp.float32),
        jax.ShapeDtypeStruct((1, 1, HIDDEN), jnp.float32),
    ),
)


@jax.jit
def kernel(input, W_word, W_pos):
    idx = input.astype(jnp.int32)
    word = _gather(idx, W_word)
    return _assemble(word, W_pos)


# speedup vs baseline: 2.7695x; 1.5947x over previous
import functools

import jax
import jax.numpy as jnp
from jax import lax
from jax.experimental import pallas as pl
from jax.experimental.pallas import tpu as pltpu
from jax.experimental.pallas import tpu_sc as plsc

L = 200
WORD_DIM = 64
POS_DIM = 64
HIDDEN = WORD_DIM + POS_DIM
ROWS_PER_WORKER = 8
NUM_CHUNKS = L // ROWS_PER_WORKER
LANE_TILE = 128

_info = plsc.get_sparse_core_info()
_NC = _info.num_cores
_NLANES = _info.num_lanes


def _make_gather():
    mesh = plsc.VectorSubcoreMesh(core_axis_name="c", subcore_axis_name="s")

    @functools.partial(
        pl.kernel,
        mesh=mesh,
        out_type=jax.ShapeDtypeStruct(
            (NUM_CHUNKS, ROWS_PER_WORKER, WORD_DIM, LANE_TILE), jnp.float32
        ),
        scratch_types=[
            pltpu.VMEM((_NLANES,), jnp.int32),
            pltpu.SemaphoreType.DMA,
        ],
    )
    def gather(idx_hbm, table_t_hbm, out_hbm, idx_v, sem):
        wid = lax.axis_index("s") * _NC + lax.axis_index("c")

        @pl.when(wid < NUM_CHUNKS)
        def _():
            base = wid * ROWS_PER_WORKER
            pltpu.sync_copy(
                idx_hbm.at[pl.ds(base, ROWS_PER_WORKER)],
                idx_v.at[pl.ds(0, ROWS_PER_WORKER)],
            )
            idx_vec = idx_v[...]
            copies = []
            for r in range(ROWS_PER_WORKER):
                col = idx_vec[r]
                col_al = pl.multiple_of(
                    (col // LANE_TILE) * LANE_TILE, LANE_TILE
                )
                copies.append(
                    pltpu.make_async_copy(
                        table_t_hbm.at[:, pl.ds(col_al, LANE_TILE)],
                        out_hbm.at[wid, r],
                        sem,
                    )
                )
            for c in copies:
                c.start()
            for c in copies:
                c.wait()

    return gather


_gather = _make_gather()


def _assemble_body(win_ref, idx_ref, pos_ref, out_ref, hid_ref):
    off = idx_ref[...] % LANE_TILE
    onehot = (
        lax.broadcasted_iota(jnp.int32, (L, LANE_TILE), 1) == off
    ).astype(jnp.float32)
    word = jnp.sum(win_ref[...] * onehot[:, None, :], axis=-1)
    rows = jnp.concatenate([word, pos_ref[...]], axis=1)
    out_ref[...] = rows
    hid_ref[...] = jnp.mean(rows, axis=0).reshape(1, 1, HIDDEN)


_assemble = pl.pallas_call(
    _assemble_body,
    out_shape=(
        jax.ShapeDtypeStruct((L, HIDDEN), jnp.float32),
        jax.ShapeDtypeStruct((1, 1, HIDDEN), jnp.float32),
    ),
)


@jax.jit
def kernel(input, W_word, W_pos):
    idx = input.astype(jnp.int32)
    windows = _gather(idx, W_word.T).reshape(L, WORD_DIM, LANE_TILE)
    return _assemble(windows, idx.reshape(L, 1), W_pos)


# speedup vs baseline: 2.7763x; 1.0024x over previous
import functools

import jax
import jax.numpy as jnp
from jax import lax
from jax.experimental import pallas as pl
from jax.experimental.pallas import tpu as pltpu
from jax.experimental.pallas import tpu_sc as plsc

L = 200
WORD_DIM = 64
POS_DIM = 64
HIDDEN = WORD_DIM + POS_DIM
ROWS_PER_WORKER = 8
NUM_CHUNKS = L // ROWS_PER_WORKER
LANE_TILE = 128

_info = plsc.get_sparse_core_info()
_NC = _info.num_cores
_NLANES = _info.num_lanes


def _make_gather():
    mesh = plsc.VectorSubcoreMesh(core_axis_name="c", subcore_axis_name="s")

    @functools.partial(
        pl.kernel,
        mesh=mesh,
        out_type=jax.ShapeDtypeStruct(
            (NUM_CHUNKS, ROWS_PER_WORKER, WORD_DIM, LANE_TILE), jnp.float32
        ),
        scratch_types=[
            pltpu.VMEM((_NLANES,), jnp.int32),
            pltpu.SemaphoreType.DMA,
        ],
    )
    def gather(idx_hbm, table_t_hbm, out_hbm, idx_v, sem):
        wid = lax.axis_index("s") + _info.num_subcores * lax.axis_index("c")

        @pl.when(wid < NUM_CHUNKS)
        def _():
            base = wid * ROWS_PER_WORKER
            pltpu.sync_copy(
                idx_hbm.at[pl.ds(base, ROWS_PER_WORKER)],
                idx_v.at[pl.ds(0, ROWS_PER_WORKER)],
            )
            idx_vec = idx_v[...]
            copies = []
            for r in range(ROWS_PER_WORKER):
                col = idx_vec[r]
                col_al = pl.multiple_of(
                    (col // LANE_TILE) * LANE_TILE, LANE_TILE
                )
                copies.append(
                    pltpu.make_async_copy(
                        table_t_hbm.at[:, pl.ds(col_al, LANE_TILE)],
                        out_hbm.at[wid, r],
                        sem,
                    )
                )
            for c in copies:
                c.start()
            for c in copies:
                c.wait()

    return gather


_gather = _make_gather()


def _assemble_body(win_ref, idx_ref, pos_ref, out_ref, hid_ref):
    off = idx_ref[...] % LANE_TILE
    onehot = (
        lax.broadcasted_iota(jnp.int32, (L, LANE_TILE), 1) == off
    ).astype(jnp.float32)
    word = jnp.sum(win_ref[...] * onehot[:, None, :], axis=-1)
    rows = jnp.concatenate([word, pos_ref[...]], axis=1)
    out_ref[...] = rows
    hid_ref[...] = jnp.mean(rows, axis=0).reshape(1, 1, HIDDEN)


_assemble = pl.pallas_call(
    _assemble_body,
    out_shape=(
        jax.ShapeDtypeStruct((L, HIDDEN), jnp.float32),
        jax.ShapeDtypeStruct((1, 1, HIDDEN), jnp.float32),
    ),
)


@jax.jit
def kernel(input, W_word, W_pos):
    idx = input.astype(jnp.int32)
    windows = _gather(idx, W_word.T).reshape(L, WORD_DIM, LANE_TILE)
    return _assemble(windows, idx.reshape(L, 1), W_pos)
